# baseline (speedup 1.0000x reference)
import jax
import jax.numpy as jnp
from jax import lax
from jax.experimental import pallas as pl
from jax.experimental.pallas import tpu as pltpu
from jax.experimental.pallas import tpu_sc as plsc

_N = 10000
_E = 640000
_D = 100
_H = 50
_C = 40
_T = 100
_DP = 128
_AW1 = 128
_AW2 = 128

_NC = 2
_NS = 16
_NW = _NC * _NS
_CH = 32
_NCHT = 640
_R = 8
_EPW = _CH * _NCHT
_EPAD = _NW * _EPW
_NP = 10240
_RPS = _NP // _NS
_NG = _CH // 16


def _make_edge_pass(nj):
  aw = _DP
  mesh = plsc.VectorSubcoreMesh(core_axis_name="c", subcore_axis_name="s")

  def body(table, idx3, rel, out, acc, rel_v, ipack, *scr):
    cid = lax.axis_index("c")
    sid = lax.axis_index("s")
    wid = sid * _NC + cid
    gbufs = scr[:_R]
    isems = scr[_R:2 * _R]
    gsems = scr[2 * _R:3 * _R]
    ssems = scr[3 * _R:4 * _R]
    g0 = gbufs[0]
    iblk = lambda k: ipack.at[pl.ds(3 * k, 3)]
    isrc = lambda k: ipack.at[3 * k]
    idst = lambda k: ipack.at[3 * k + 1]

    for c0 in range(_R - 2):
      pltpu.async_copy(idx3.at[wid, c0], iblk(c0), isems[c0])
    pltpu.sync_copy(rel, rel_v)

    def zrow(r, carry):
      for j in range(8):
        g0[r, pl.ds(j * 16, 16)] = jnp.zeros((16,), jnp.float32)
      return carry
    lax.fori_loop(0, _CH, zrow, 0)
    row0 = sid * _RPS
    for b in range(_RPS // _CH):
      pltpu.sync_copy(g0, acc.at[pl.ds(row0 + b * _CH, _CH)])
    plsc.subcore_barrier()

    for c0 in range(_R - 2):
      pltpu.make_async_copy(idx3.at[wid, c0], iblk(c0), isems[c0]).wait()
      pltpu.async_copy(table.at[isrc(c0)], gbufs[c0], gsems[c0])

    def step(c, k):
      g = gbufs[k]

      pltpu.make_async_copy(table.at[isrc(k)], g, gsems[k]).wait()

      @pl.when(c >= 2)
      def _():
        pltpu.make_async_copy(g, acc.at[idst(k)], ssems[(k + _R - 2) % _R]
                              ).wait()

      @pl.when(c + _R - 2 < _NCHT)
      def _():
        pltpu.async_copy(idx3.at[wid, c + _R - 2], iblk((k + _R - 2) % _R),
                         isems[(k + _R - 2) % _R])

      @plsc.parallel_loop(0, _NG, step=1, carry=jnp.int32(0))
      def group(gr, carry):
        tv = ipack[3 * k + 2, pl.ds(gr * 16, 16)]
        for lane in range(16):
          t = tv[lane]
          e = gr * 16 + lane
          for j in range(nj):
            sl = pl.ds(j * 16, 16)
            g[e, sl] = g[e, sl] * rel_v[t, sl]
        return carry

      pltpu.async_copy(g, acc.at[idst(k)], ssems[k], add=True)

      @pl.when(c + _R - 2 < _NCHT)
      def _():
        kn = (k + _R - 2) % _R
        pltpu.make_async_copy(idx3.at[wid, c + _R - 2], iblk(kn),
                              isems[kn]).wait()
        pltpu.async_copy(table.at[isrc(kn)], gbufs[kn], gsems[kn])

    def ring(q, carry):
      for k in range(_R):
        step(_R * q + k, k)
      return carry
    lax.fori_loop(0, _NCHT // _R, ring, 0)

    for c in (_NCHT - 2, _NCHT - 1):
      k = c % _R
      pltpu.make_async_copy(gbufs[k], acc.at[idst(k)], ssems[k]).wait()
    plsc.subcore_barrier()
    pltpu.sync_copy(acc.at[pl.ds(row0, _RPS)], out.at[cid, pl.ds(row0, _RPS)])

  return pl.kernel(
      body,
      out_type=jax.ShapeDtypeStruct((_NC, _NP, aw), jnp.float32),
      mesh=mesh,
      scratch_types=(
          [pltpu.VMEM_SHARED((_NP, aw), jnp.float32),
           pltpu.VMEM((_T, aw), jnp.float32),
           pltpu.VMEM((3 * _R, _CH), jnp.int32)]
          + [pltpu.VMEM((_CH, _DP), jnp.float32)] * _R
          + [pltpu.SemaphoreType.DMA] * (3 * _R)
      ),
  )


_edge_pass_1 = _make_edge_pass(7)
_edge_pass_2 = _make_edge_pass(4)


def _prep_body(x_ref, tn_ref, rb_ref, xp_ref, r1_ref):
  tn = tn_ref[0]
  rows = lax.broadcasted_iota(jnp.int32, (_N, 1), 0)
  xm = jnp.where(rows == tn, 0.0, x_ref[...])
  xp_ref[:, :_D] = xm
  xp_ref[:, _D:_D + 1] = jnp.ones((_N, 1), jnp.float32)
  xp_ref[:, _D + 1:] = jnp.zeros((_N, _DP - _D - 1), jnp.float32)
  rb = rb_ref[...]
  r1_ref[: _T // 2, :_D] = rb
  r1_ref[_T // 2:, :_D] = rb
  r1_ref[:, _D:_D + 1] = jnp.ones((_T, 1), jnp.float32)
  r1_ref[:, _D + 1:] = jnp.zeros((_T, _AW1 - _D - 1), jnp.float32)


def _dense1_body(p1_ref, xp_ref, w1m_ref, w1r_ref, b1_ref, r1_ref, rw_ref,
                 h_ref, r2_ref):
  p = p1_ref[0, :_N] + p1_ref[1, :_N]
  degc = jnp.maximum(p[:, _D:_D + 1], 1.0)
  feat = p[:, :_D] / degc
  h = (jnp.dot(feat, w1m_ref[...], preferred_element_type=jnp.float32)
       + jnp.dot(xp_ref[:, :_D], w1r_ref[...],
                 preferred_element_type=jnp.float32)
       + b1_ref[...])
  h_ref[:, :_H] = h
  h_ref[:, _H:_H + 1] = jnp.ones((_N, 1), jnp.float32)
  h_ref[:, _H + 1:] = jnp.zeros((_N, _DP - _H - 1), jnp.float32)
  r2 = jnp.dot(r1_ref[:, :_D], rw_ref[...], preferred_element_type=jnp.float32)
  r2_ref[:, :_H] = r2
  r2_ref[:, _H:_H + 1] = jnp.ones((_T, 1), jnp.float32)
  r2_ref[:, _H + 1:] = jnp.zeros((_T, _AW2 - _H - 1), jnp.float32)


def _dense2_body(p2_ref, hp_ref, w2m_ref, w2r_ref, b2_ref, o_ref):
  p = p2_ref[0, :_N] + p2_ref[1, :_N]
  degc = jnp.maximum(p[:, _H:_H + 1], 1.0)
  feat = p[:, :_H] / degc
  out = (jnp.dot(feat, w2m_ref[...], preferred_element_type=jnp.float32)
         + jnp.dot(hp_ref[:, :_H], w2r_ref[...],
                   preferred_element_type=jnp.float32)
         + b2_ref[...])
  m = jnp.max(out, axis=1, keepdims=True)
  s = out - m
  lse = jnp.log(jnp.sum(jnp.exp(s), axis=1, keepdims=True))
  o_ref[...] = s - lse


def kernel(x, rel_base, rel_weight, W1_msg, W1_root, b1, W2_msg, W2_root, b2,
           edge_index, edge_type, target_node):
  tn = jnp.asarray(target_node, jnp.int32).reshape(1)

  npad = _EPAD - _E
  src_p = jnp.concatenate([edge_index[0], jnp.zeros((npad,), jnp.int32)])
  dst_p = jnp.concatenate(
      [edge_index[1], jnp.full((npad,), _NP - 1, jnp.int32)])
  et_p = jnp.concatenate([edge_type, jnp.zeros((npad,), jnp.int32)])
  idx3 = jnp.stack(
      [a.reshape(_NW, _NCHT, _CH) for a in (src_p, dst_p, et_p)], axis=2)

  xp, r1p = pl.pallas_call(
      _prep_body,
      out_shape=(
          jax.ShapeDtypeStruct((_N, _DP), jnp.float32),
          jax.ShapeDtypeStruct((_T, _AW1), jnp.float32),
      ),
      in_specs=[
          pl.BlockSpec(memory_space=pltpu.VMEM),
          pl.BlockSpec(memory_space=pltpu.SMEM),
          pl.BlockSpec(memory_space=pltpu.VMEM),
      ],
  )(x, tn, rel_base)

  p1 = _edge_pass_1(xp, idx3, r1p)

  hp, r2p = pl.pallas_call(
      _dense1_body,
      out_shape=(
          jax.ShapeDtypeStruct((_N, _DP), jnp.float32),
          jax.ShapeDtypeStruct((_T, _AW2), jnp.float32),
      ),
  )(p1, xp, W1_msg, W1_root, b1.reshape(1, _H), r1p, rel_weight)

  p2 = _edge_pass_2(hp, idx3, r2p)

  out = pl.pallas_call(
      _dense2_body,
      out_shape=jax.ShapeDtypeStruct((_N, _C), jnp.float32),
  )(p2, hp, W2_msg, W2_root, b2.reshape(1, _C))
  return out


# speedup vs baseline: 1.1197x; 1.1197x over previous
import jax
import jax.numpy as jnp
from jax import lax
from jax.experimental import pallas as pl
from jax.experimental.pallas import tpu as pltpu
from jax.experimental.pallas import tpu_sc as plsc

_N = 10000
_E = 640000
_D = 100
_H = 50
_C = 40
_T = 100
_DP = 128
_AW1 = 128
_AW2 = 128

_NC = 2
_NS = 16
_NW = _NC * _NS
_CH = 64
_NCHT = 320
_EPW = _CH * _NCHT
_EPAD = _NW * _EPW
_NP = 10240
_RPS = _NP // _NS
_NG = _CH // 16


def _make_edge_pass(nj):
  aw = _DP
  mesh = plsc.VectorSubcoreMesh(core_axis_name="c", subcore_axis_name="s")

  def body(table, idx3, rel, out, acc, rel_v,
           i0, i1, i2, i3, g0, g1, g2, g3,
           is0, is1, is2, is3, gsem0, gsem1, gsem2, gsem3,
           ssem0, ssem1, ssem2, ssem3):
    cid = lax.axis_index("c")
    sid = lax.axis_index("s")
    wid = sid * _NC + cid
    ibufs = (i0, i1, i2, i3)
    isems = (is0, is1, is2, is3)
    gbufs = (g0, g1, g2, g3)
    gsems = (gsem0, gsem1, gsem2, gsem3)
    ssems = (ssem0, ssem1, ssem2, ssem3)

    pltpu.async_copy(idx3.at[wid, 0], i0, is0)
    pltpu.async_copy(idx3.at[wid, 1], i1, is1)
    pltpu.sync_copy(rel, rel_v)

    def zrow(r, carry):
      for j in range(8):
        g0[r, pl.ds(j * 16, 16)] = jnp.zeros((16,), jnp.float32)
      return carry
    lax.fori_loop(0, _CH, zrow, 0)
    row0 = sid * _RPS
    for b in range(_RPS // _CH):
      pltpu.sync_copy(g0, acc.at[pl.ds(row0 + b * _CH, _CH)])
    plsc.subcore_barrier()

    for c0 in range(2):
      pltpu.make_async_copy(idx3.at[wid, c0], ibufs[c0], isems[c0]).wait()
      pltpu.async_copy(table.at[ibufs[c0].at[0]], gbufs[c0], gsems[c0])

    def step(c, k):
      ib = ibufs[k]
      g = gbufs[k]

      pltpu.make_async_copy(table.at[ib.at[0]], g, gsems[k]).wait()

      @pl.when(c >= 2)
      def _():
        pltpu.make_async_copy(g, acc.at[ib.at[1]], ssems[(k + 2) % 4]
                              ).wait()

      @pl.when(c + 2 < _NCHT)
      def _():
        pltpu.async_copy(idx3.at[wid, c + 2], ibufs[(k + 2) % 4],
                         isems[(k + 2) % 4])

      @plsc.parallel_loop(0, _NG, step=1, carry=jnp.int32(0))
      def group(gr, carry):
        tv = ib[2, pl.ds(gr * 16, 16)]
        for lane in range(16):
          t = tv[lane]
          e = gr * 16 + lane
          for j in range(nj):
            sl = pl.ds(j * 16, 16)
            g[e, sl] = g[e, sl] * rel_v[t, sl]
        return carry

      pltpu.async_copy(g, acc.at[ib.at[1]], ssems[k], add=True)

      @pl.when(c + 2 < _NCHT)
      def _():
        ibn = ibufs[(k + 2) % 4]
        pltpu.make_async_copy(idx3.at[wid, c + 2], ibn, isems[(k + 2) % 4]
                              ).wait()
        pltpu.async_copy(table.at[ibn.at[0]], gbufs[(k + 2) % 4],
                         gsems[(k + 2) % 4])

    def quad(q, carry):
      for k in range(4):
        step(4 * q + k, k)
      return carry
    lax.fori_loop(0, _NCHT // 4, quad, 0)

    pltpu.make_async_copy(g2, acc.at[i2.at[1]], ssem2).wait()
    pltpu.make_async_copy(g3, acc.at[i3.at[1]], ssem3).wait()
    plsc.subcore_barrier()
    pltpu.sync_copy(acc.at[pl.ds(row0, _RPS)], out.at[cid, pl.ds(row0, _RPS)])

  return pl.kernel(
      body,
      out_type=jax.ShapeDtypeStruct((_NC, _NP, aw), jnp.float32),
      mesh=mesh,
      scratch_types=[
          pltpu.VMEM_SHARED((_NP, aw), jnp.float32),
          pltpu.VMEM((_T, aw), jnp.float32),
          pltpu.VMEM((3, _CH), jnp.int32),
          pltpu.VMEM((3, _CH), jnp.int32),
          pltpu.VMEM((3, _CH), jnp.int32),
          pltpu.VMEM((3, _CH), jnp.int32),
          pltpu.VMEM((_CH, _DP), jnp.float32),
          pltpu.VMEM((_CH, _DP), jnp.float32),
          pltpu.VMEM((_CH, _DP), jnp.float32),
          pltpu.VMEM((_CH, _DP), jnp.float32),
          pltpu.SemaphoreType.DMA,
          pltpu.SemaphoreType.DMA,
          pltpu.SemaphoreType.DMA,
          pltpu.SemaphoreType.DMA,
          pltpu.SemaphoreType.DMA,
          pltpu.SemaphoreType.DMA,
          pltpu.SemaphoreType.DMA,
          pltpu.SemaphoreType.DMA,
          pltpu.SemaphoreType.DMA,
          pltpu.SemaphoreType.DMA,
          pltpu.SemaphoreType.DMA,
          pltpu.SemaphoreType.DMA,
      ],
  )


_edge_pass_1 = _make_edge_pass(7)
_edge_pass_2 = _make_edge_pass(4)


def _prep_body(x_ref, tn_ref, rb_ref, xp_ref, r1_ref):
  tn = tn_ref[0]
  rows = lax.broadcasted_iota(jnp.int32, (_N, 1), 0)
  xm = jnp.where(rows == tn, 0.0, x_ref[...])
  xp_ref[:, :_D] = xm
  xp_ref[:, _D:_D + 1] = jnp.ones((_N, 1), jnp.float32)
  xp_ref[:, _D + 1:] = jnp.zeros((_N, _DP - _D - 1), jnp.float32)
  rb = rb_ref[...]
  r1_ref[: _T // 2, :_D] = rb
  r1_ref[_T // 2:, :_D] = rb
  r1_ref[:, _D:_D + 1] = jnp.ones((_T, 1), jnp.float32)
  r1_ref[:, _D + 1:] = jnp.zeros((_T, _AW1 - _D - 1), jnp.float32)


def _dense1_body(p1_ref, xp_ref, w1m_ref, w1r_ref, b1_ref, r1_ref, rw_ref,
                 h_ref, r2_ref):
  p = p1_ref[0, :_N] + p1_ref[1, :_N]
  degc = jnp.maximum(p[:, _D:_D + 1], 1.0)
  feat = p[:, :_D] / degc
  h = (jnp.dot(feat, w1m_ref[...], preferred_element_type=jnp.float32)
       + jnp.dot(xp_ref[:, :_D], w1r_ref[...],
                 preferred_element_type=jnp.float32)
       + b1_ref[...])
  h_ref[:, :_H] = h
  h_ref[:, _H:_H + 1] = jnp.ones((_N, 1), jnp.float32)
  h_ref[:, _H + 1:] = jnp.zeros((_N, _DP - _H - 1), jnp.float32)
  r2 = jnp.dot(r1_ref[:, :_D], rw_ref[...], preferred_element_type=jnp.float32)
  r2_ref[:, :_H] = r2
  r2_ref[:, _H:_H + 1] = jnp.ones((_T, 1), jnp.float32)
  r2_ref[:, _H + 1:] = jnp.zeros((_T, _AW2 - _H - 1), jnp.float32)


def _dense2_body(p2_ref, hp_ref, w2m_ref, w2r_ref, b2_ref, o_ref):
  p = p2_ref[0, :_N] + p2_ref[1, :_N]
  degc = jnp.maximum(p[:, _H:_H + 1], 1.0)
  feat = p[:, :_H] / degc
  out = (jnp.dot(feat, w2m_ref[...], preferred_element_type=jnp.float32)
         + jnp.dot(hp_ref[:, :_H], w2r_ref[...],
                   preferred_element_type=jnp.float32)
         + b2_ref[...])
  m = jnp.max(out, axis=1, keepdims=True)
  s = out - m
  lse = jnp.log(jnp.sum(jnp.exp(s), axis=1, keepdims=True))
  o_ref[...] = s - lse


def kernel(x, rel_base, rel_weight, W1_msg, W1_root, b1, W2_msg, W2_root, b2,
           edge_index, edge_type, target_node):
  tn = jnp.asarray(target_node, jnp.int32).reshape(1)

  npad = _EPAD - _E
  src_p = jnp.concatenate([edge_index[0], jnp.zeros((npad,), jnp.int32)])
  dst_p = jnp.concatenate(
      [edge_index[1], jnp.full((npad,), _NP - 1, jnp.int32)])
  et_p = jnp.concatenate([edge_type, jnp.zeros((npad,), jnp.int32)])
  idx3 = jnp.stack(
      [a.reshape(_NW, _NCHT, _CH) for a in (src_p, dst_p, et_p)], axis=2)

  xp, r1p = pl.pallas_call(
      _prep_body,
      out_shape=(
          jax.ShapeDtypeStruct((_N, _DP), jnp.float32),
          jax.ShapeDtypeStruct((_T, _AW1), jnp.float32),
      ),
      in_specs=[
          pl.BlockSpec(memory_space=pltpu.VMEM),
          pl.BlockSpec(memory_space=pltpu.SMEM),
          pl.BlockSpec(memory_space=pltpu.VMEM),
      ],
  )(x, tn, rel_base)

  p1 = _edge_pass_1(xp, idx3, r1p)

  hp, r2p = pl.pallas_call(
      _dense1_body,
      out_shape=(
          jax.ShapeDtypeStruct((_N, _DP), jnp.float32),
          jax.ShapeDtypeStruct((_T, _AW2), jnp.float32),
      ),
  )(p1, xp, W1_msg, W1_root, b1.reshape(1, _H), r1p, rel_weight)

  p2 = _edge_pass_2(hp, idx3, r2p)

  out = pl.pallas_call(
      _dense2_body,
      out_shape=jax.ShapeDtypeStruct((_N, _C), jnp.float32),
  )(p2, hp, W2_msg, W2_root, b2.reshape(1, _C))
  return out
